# Initial kernel scaffold; baseline (speedup 1.0000x reference)
import jax
import jax.numpy as jnp
from jax.experimental import pallas as pl


def kernel(x, edge_index, batch, params):
    raise NotImplementedError("write your pallas kernel here")


# speedup vs baseline: 4.3696x; 4.3696x over previous
import functools

import jax
import jax.numpy as jnp
from jax import lax
from jax.experimental import pallas as pl
from jax.experimental.pallas import tpu as pltpu
from jax.experimental.pallas import tpu_sc as plsc

N = 10000
E = 320000
D = 128
H = 64
T = 64
G = 64

NC = 2
NS = 16
NW = NC * NS

CHUNK = 128
CH = 80
EPT = CH * CHUNK
EP = NW * EPT
NP = 10112
RPT = NP // NS


def _sc_segsum_body(h_hbm, srcs_hbm, dsts_hbm, zeros_hbm, out_hbm,
                    src_v, dst_v, rows0, rows1, agg_sh, sem0, sem1):
    c = lax.axis_index("c")
    s = lax.axis_index("s")
    wid = s * NC + c

    pltpu.sync_copy(srcs_hbm.at[wid], src_v)
    pltpu.sync_copy(dsts_hbm.at[wid], dst_v)

    r0 = s * RPT
    pltpu.sync_copy(zeros_hbm, agg_sh.at[pl.ds(r0, RPT)])
    plsc.subcore_barrier()

    bufs = (rows0, rows1)
    sems = (sem0, sem1)

    def start(j, b):
        pltpu.async_copy(h_hbm.at[src_v.at[j]], bufs[b], sems[b])

    def finish(j, b):
        pltpu.make_async_copy(h_hbm.at[src_v.at[j]], bufs[b], sems[b]).wait()
        pltpu.sync_copy(bufs[b], agg_sh.at[dst_v.at[j]], add=True)

    start(0, 0)

    @pl.loop(0, CH, step=2)
    def _(jj):
        start(jj + 1, 1)
        finish(jj, 0)

        @pl.when(jj + 2 < CH)
        def _():
            start(jj + 2, 0)

        finish(jj + 1, 1)

    plsc.subcore_barrier()
    pltpu.sync_copy(agg_sh.at[pl.ds(r0, RPT)], out_hbm.at[c, pl.ds(r0, RPT)])


def _sc_segsum(h, srcs, dsts, zeros):
    mesh = plsc.VectorSubcoreMesh(core_axis_name="c", subcore_axis_name="s")
    return pl.kernel(
        _sc_segsum_body,
        out_type=jax.ShapeDtypeStruct((NC, NP, H), jnp.float32),
        mesh=mesh,
        scratch_types=[
            pltpu.VMEM((CH, CHUNK), jnp.int32),
            pltpu.VMEM((CH, CHUNK), jnp.int32),
            pltpu.VMEM((CHUNK, H), jnp.float32),
            pltpu.VMEM((CHUNK, H), jnp.float32),
            pltpu.VMEM_SHARED((NP, H), jnp.float32),
            pltpu.SemaphoreType.DMA,
            pltpu.SemaphoreType.DMA,
        ],
        compiler_params=pltpu.CompilerParams(use_tc_tiling_on_sc=False),
        name="sc_segsum",
    )(h, srcs, dsts, zeros)


def _bn_relu(z, g, b):
    m = jnp.mean(z, axis=0, keepdims=True)
    d = z - m
    v = jnp.mean(d * d, axis=0, keepdims=True)
    return jnp.maximum(d * lax.rsqrt(v + 1e-5) * g + b, 0.0)


def _stage_first_body(x_ref, w1_ref, w2_ref, wl_ref, vec_ref, h_ref, t_ref):
    x = x_ref[...]
    z = jnp.dot(x, w1_ref[...], preferred_element_type=jnp.float32)
    h = _bn_relu(z + vec_ref[0:1, :], vec_ref[1:2, :], vec_ref[2:3, :])
    z2 = jnp.dot(h, w2_ref[...], preferred_element_type=jnp.float32)
    h2 = _bn_relu(z2 + vec_ref[3:4, :], vec_ref[4:5, :], vec_ref[5:6, :])
    h_ref[...] = h2
    t_ref[...] = jnp.dot(h2, wl_ref[...], preferred_element_type=jnp.float32) + vec_ref[6:7, :]


def _stage_conv_body(hin_ref, a_ref, w1_ref, w2_ref, wl_ref, vec_ref, h_ref, t_ref):
    x = hin_ref[...] + a_ref[0, 0:N, :] + a_ref[1, 0:N, :]
    z = jnp.dot(x, w1_ref[...], preferred_element_type=jnp.float32)
    h = _bn_relu(z + vec_ref[0:1, :], vec_ref[1:2, :], vec_ref[2:3, :])
    z2 = jnp.dot(h, w2_ref[...], preferred_element_type=jnp.float32)
    h2 = _bn_relu(z2 + vec_ref[3:4, :], vec_ref[4:5, :], vec_ref[5:6, :])
    h_ref[...] = h2
    t_ref[...] = jnp.dot(h2, wl_ref[...], preferred_element_type=jnp.float32) + vec_ref[6:7, :]


_STAGE_OUT = (
    jax.ShapeDtypeStruct((N, H), jnp.float32),
    jax.ShapeDtypeStruct((N, T), jnp.float32),
)


def _pack_vecs(mlp, lin):
    return jnp.stack([
        mlp["lin1"]["b"], mlp["bn1"]["g"], mlp["bn1"]["b"],
        mlp["lin2"]["b"], mlp["bn2"]["g"], mlp["bn2"]["b"],
        lin["b"], jnp.zeros((H,), jnp.float32),
    ])


def _stage_first(x, mlp, lin):
    return pl.pallas_call(_stage_first_body, out_shape=_STAGE_OUT)(
        x, mlp["lin1"]["W"].T, mlp["lin2"]["W"].T, lin["W"].T,
        _pack_vecs(mlp, lin))


def _stage_conv(hin, agg, mlp, lin):
    return pl.pallas_call(_stage_conv_body, out_shape=_STAGE_OUT)(
        hin, agg, mlp["lin1"]["W"].T, mlp["lin2"]["W"].T, lin["W"].T,
        _pack_vecs(mlp, lin))


def _pool_body(t_ref, b_ref, o_ref):
    gblk = pl.program_id(0)
    t = t_ref[...]
    bids = b_ref[...]
    for k in range(8):
        mask = bids == gblk * 8 + k
        o_ref[k:k + 1, :] = jnp.max(jnp.where(mask, t, -jnp.inf), axis=0,
                                    keepdims=True)


def _pool(t, bids):
    return pl.pallas_call(
        _pool_body,
        grid=(G // 8,),
        in_specs=[
            pl.BlockSpec((N, T), lambda g: (0, 0)),
            pl.BlockSpec((N, 1), lambda g: (0, 0)),
        ],
        out_specs=pl.BlockSpec((8, T), lambda g: (g, 0)),
        out_shape=jax.ShapeDtypeStruct((G, T), jnp.float32),
    )(t, bids)


def kernel(x, edge_index, batch, params):
    ei = edge_index.astype(jnp.int32)
    pad = EP - E
    src = jnp.concatenate([ei[0], jnp.zeros((pad,), jnp.int32)])
    dst = jnp.concatenate(
        [ei[1], N + (jnp.arange(pad, dtype=jnp.int32) % (NP - N))])
    srcs = src.reshape(NW, CH, CHUNK)
    dsts = dst.reshape(NW, CH, CHUNK)
    zeros = jnp.zeros((RPT, H), jnp.float32)
    bids = batch.astype(jnp.int32).reshape(N, 1)

    p = params
    h0, t0 = _stage_first(x, p["first_h"], p["linears"][0])
    out = _pool(t0, bids)
    h = h0
    for i in range(2):
        agg = _sc_segsum(h, srcs, dsts, zeros)
        h, t = _stage_conv(h, agg, p["convs"][i], p["linears"][i + 1])
        out = out + _pool(t, bids)
    return (out, h)


# speedup vs baseline: 6.5236x; 1.4930x over previous
import functools

import jax
import jax.numpy as jnp
from jax import lax
from jax.experimental import pallas as pl
from jax.experimental.pallas import tpu as pltpu
from jax.experimental.pallas import tpu_sc as plsc

N = 10000
E = 320000
D = 128
H = 64
T = 64
G = 64

NC = 2
NS = 16
NW = NC * NS

CHUNK = 128
CH = 80
EPT = CH * CHUNK
EP = NW * EPT
NP = 10112
RPT = NP // NS


NB = 4
LA = 2


def _sc_segsum_body(h_hbm, srcs_hbm, dsts_hbm, zeros_hbm, out_hbm,
                    src_v, dst_v, bufs, gsems, ssems, agg_sh):
    c = lax.axis_index("c")
    s = lax.axis_index("s")
    wid = s * NC + c

    pltpu.sync_copy(srcs_hbm.at[wid], src_v)
    pltpu.sync_copy(dsts_hbm.at[wid], dst_v)

    r0 = s * RPT
    pltpu.sync_copy(zeros_hbm.at[pl.ds(r0, RPT)], agg_sh.at[pl.ds(r0, RPT)])
    plsc.subcore_barrier()

    def start_gather(j, b):
        pltpu.async_copy(h_hbm.at[src_v.at[j]], bufs[b], gsems[b])

    def wait_gather(j, b):
        pltpu.make_async_copy(h_hbm.at[src_v.at[j]], bufs[b], gsems[b]).wait()

    def start_scatter(j, b):
        pltpu.async_copy(bufs[b], agg_sh.at[dst_v.at[j]], ssems[b], add=True)

    def wait_scatter(j, b):
        pltpu.make_async_copy(bufs[b], agg_sh.at[dst_v.at[j]], ssems[b]).wait()

    for j in range(LA):
        start_gather(j, j % NB)

    @pl.loop(0, CH, step=NB)
    def _(jj):
        for b in range(NB):
            j = jj + b
            bg = (b + LA) % NB

            @pl.when(j + LA < CH)
            def _():
                @pl.when(j + LA - NB >= 0)
                def _():
                    wait_scatter(j + LA - NB, bg)
                start_gather(j + LA, bg)

            wait_gather(j, b)
            start_scatter(j, b)

    for k in range(CH - NB, CH):
        wait_scatter(k, k % NB)

    plsc.subcore_barrier()
    pltpu.sync_copy(agg_sh.at[pl.ds(r0, RPT)], out_hbm.at[c, pl.ds(r0, RPT)])


def _sc_segsum(h, srcs, dsts, zeros):
    mesh = plsc.VectorSubcoreMesh(core_axis_name="c", subcore_axis_name="s")
    return pl.kernel(
        _sc_segsum_body,
        out_type=jax.ShapeDtypeStruct((NC, NP, H), jnp.float32),
        mesh=mesh,
        scratch_types=[
            pltpu.VMEM((CH, CHUNK), jnp.int32),
            pltpu.VMEM((CH, CHUNK), jnp.int32),
            [pltpu.VMEM((CHUNK, H), jnp.float32) for _ in range(NB)],
            [pltpu.SemaphoreType.DMA for _ in range(NB)],
            [pltpu.SemaphoreType.DMA for _ in range(NB)],
            pltpu.VMEM_SHARED((NP, H), jnp.float32),
        ],
        compiler_params=pltpu.CompilerParams(use_tc_tiling_on_sc=False),
        name="sc_segsum",
    )(h, srcs, dsts, zeros)


def _bn_relu(z, g, b):
    m = jnp.mean(z, axis=0, keepdims=True)
    d = z - m
    v = jnp.mean(d * d, axis=0, keepdims=True)
    return jnp.maximum(d * lax.rsqrt(v + 1e-5) * g + b, 0.0)


def _stage_first_body(x_ref, w1_ref, w2_ref, wl_ref, vec_ref, h_ref, t_ref):
    x = x_ref[...]
    z = jnp.dot(x, w1_ref[...], preferred_element_type=jnp.float32)
    h = _bn_relu(z + vec_ref[0:1, :], vec_ref[1:2, :], vec_ref[2:3, :])
    z2 = jnp.dot(h, w2_ref[...], preferred_element_type=jnp.float32)
    h2 = _bn_relu(z2 + vec_ref[3:4, :], vec_ref[4:5, :], vec_ref[5:6, :])
    h_ref[...] = h2
    t_ref[...] = jnp.dot(h2, wl_ref[...], preferred_element_type=jnp.float32) + vec_ref[6:7, :]


def _stage_conv_body(hin_ref, a_ref, w1_ref, w2_ref, wl_ref, vec_ref, h_ref, t_ref):
    x = hin_ref[...] + a_ref[0, 0:N, :] + a_ref[1, 0:N, :]
    z = jnp.dot(x, w1_ref[...], preferred_element_type=jnp.float32)
    h = _bn_relu(z + vec_ref[0:1, :], vec_ref[1:2, :], vec_ref[2:3, :])
    z2 = jnp.dot(h, w2_ref[...], preferred_element_type=jnp.float32)
    h2 = _bn_relu(z2 + vec_ref[3:4, :], vec_ref[4:5, :], vec_ref[5:6, :])
    h_ref[...] = h2
    t_ref[...] = jnp.dot(h2, wl_ref[...], preferred_element_type=jnp.float32) + vec_ref[6:7, :]


_STAGE_OUT = (
    jax.ShapeDtypeStruct((N, H), jnp.float32),
    jax.ShapeDtypeStruct((N, T), jnp.float32),
)


def _pack_vecs(mlp, lin):
    return jnp.stack([
        mlp["lin1"]["b"], mlp["bn1"]["g"], mlp["bn1"]["b"],
        mlp["lin2"]["b"], mlp["bn2"]["g"], mlp["bn2"]["b"],
        lin["b"], jnp.zeros((H,), jnp.float32),
    ])


def _stage_first(x, mlp, lin):
    return pl.pallas_call(_stage_first_body, out_shape=_STAGE_OUT)(
        x, mlp["lin1"]["W"].T, mlp["lin2"]["W"].T, lin["W"].T,
        _pack_vecs(mlp, lin))


def _stage_conv(hin, agg, mlp, lin):
    return pl.pallas_call(_stage_conv_body, out_shape=_STAGE_OUT)(
        hin, agg, mlp["lin1"]["W"].T, mlp["lin2"]["W"].T, lin["W"].T,
        _pack_vecs(mlp, lin))


def _pool_body(t_ref, b_ref, o_ref):
    gblk = pl.program_id(0)
    t = t_ref[...]
    bids = b_ref[...]
    for k in range(8):
        mask = bids == gblk * 8 + k
        o_ref[k:k + 1, :] = jnp.max(jnp.where(mask, t, -jnp.inf), axis=0,
                                    keepdims=True)


def _pool(t, bids):
    return pl.pallas_call(
        _pool_body,
        grid=(G // 8,),
        in_specs=[
            pl.BlockSpec((N, T), lambda g: (0, 0)),
            pl.BlockSpec((N, 1), lambda g: (0, 0)),
        ],
        out_specs=pl.BlockSpec((8, T), lambda g: (g, 0)),
        out_shape=jax.ShapeDtypeStruct((G, T), jnp.float32),
    )(t, bids)


def kernel(x, edge_index, batch, params):
    ei = edge_index.astype(jnp.int32)
    pad = EP - E
    parange = jnp.arange(pad, dtype=jnp.int32)
    src = jnp.concatenate([ei[0], parange % N])
    dst = jnp.concatenate([ei[1], N + parange % (NP - N)])
    srcs = src.reshape(NW, CH, CHUNK)
    dsts = dst.reshape(NW, CH, CHUNK)
    zeros = jnp.zeros((NP, H), jnp.float32)
    bids = batch.astype(jnp.int32).reshape(N, 1)

    p = params
    h0, t0 = _stage_first(x, p["first_h"], p["linears"][0])
    out = _pool(t0, bids)
    h = h0
    for i in range(2):
        agg = _sc_segsum(h, srcs, dsts, zeros)
        h, t = _stage_conv(h, agg, p["convs"][i], p["linears"][i + 1])
        out = out + _pool(t, bids)
    return (out, h)


# speedup vs baseline: 7.2043x; 1.1043x over previous
import functools

import jax
import jax.numpy as jnp
from jax import lax
from jax.experimental import pallas as pl
from jax.experimental.pallas import tpu as pltpu
from jax.experimental.pallas import tpu_sc as plsc

N = 10000
E = 320000
D = 128
H = 64
T = 64
G = 64

NC = 2
NS = 16
NW = NC * NS

CHUNK = 128
CH = 80
EPT = CH * CHUNK
EP = NW * EPT
NP = 10112
RPT = NP // NS


NB = 4
LA = 2


def _sc_segsum_body(h_hbm, srcs_hbm, dsts_hbm, zeros_hbm, out_hbm,
                    src_v, dst_v, bufs, gsems, ssems, agg_sh):
    c = lax.axis_index("c")
    s = lax.axis_index("s")
    wid = s * NC + c

    pltpu.sync_copy(srcs_hbm.at[wid], src_v)
    pltpu.sync_copy(dsts_hbm.at[wid], dst_v)

    r0 = s * RPT
    pltpu.sync_copy(zeros_hbm.at[pl.ds(r0, RPT)], agg_sh.at[pl.ds(r0, RPT)])
    plsc.subcore_barrier()

    def start_gather(j, b):
        pltpu.async_copy(h_hbm.at[src_v.at[j]], bufs[b], gsems[b])

    def wait_gather(j, b):
        pltpu.make_async_copy(h_hbm.at[src_v.at[j]], bufs[b], gsems[b]).wait()

    def start_scatter(j, b):
        pltpu.async_copy(bufs[b], agg_sh.at[dst_v.at[j]], ssems[b], add=True)

    def wait_scatter(j, b):
        pltpu.make_async_copy(bufs[b], agg_sh.at[dst_v.at[j]], ssems[b]).wait()

    for j in range(LA):
        start_gather(j, j % NB)

    @pl.loop(0, CH, step=NB)
    def _(jj):
        for b in range(NB):
            j = jj + b
            bg = (b + LA) % NB

            @pl.when(j + LA < CH)
            def _():
                @pl.when(j + LA - NB >= 0)
                def _():
                    wait_scatter(j + LA - NB, bg)
                start_gather(j + LA, bg)

            wait_gather(j, b)
            start_scatter(j, b)

    for k in range(CH - NB, CH):
        wait_scatter(k, k % NB)

    plsc.subcore_barrier()
    pltpu.sync_copy(agg_sh.at[pl.ds(r0, RPT)], out_hbm.at[c, pl.ds(r0, RPT)])


def _sc_segsum(h, srcs, dsts, zeros):
    mesh = plsc.VectorSubcoreMesh(core_axis_name="c", subcore_axis_name="s")
    return pl.kernel(
        _sc_segsum_body,
        out_type=jax.ShapeDtypeStruct((NC, NP, H), jnp.float32),
        mesh=mesh,
        scratch_types=[
            pltpu.VMEM((CH, CHUNK), jnp.int32),
            pltpu.VMEM((CH, CHUNK), jnp.int32),
            [pltpu.VMEM((CHUNK, H), jnp.float32) for _ in range(NB)],
            [pltpu.SemaphoreType.DMA for _ in range(NB)],
            [pltpu.SemaphoreType.DMA for _ in range(NB)],
            pltpu.VMEM_SHARED((NP, H), jnp.float32),
        ],
        compiler_params=pltpu.CompilerParams(use_tc_tiling_on_sc=False),
        name="sc_segsum",
    )(h, srcs, dsts, zeros)


def _bn_relu(z, g, b):
    m = jnp.mean(z, axis=0, keepdims=True)
    d = z - m
    v = jnp.mean(d * d, axis=0, keepdims=True)
    return jnp.maximum(d * lax.rsqrt(v + 1e-5) * g + b, 0.0)


def _stage_first_body(x_ref, w1_ref, w2_ref, wl_ref, vec_ref, h_ref, t_ref):
    x = x_ref[...]
    z = jnp.dot(x, w1_ref[...], preferred_element_type=jnp.float32)
    h = _bn_relu(z + vec_ref[0:1, :], vec_ref[1:2, :], vec_ref[2:3, :])
    z2 = jnp.dot(h, w2_ref[...], preferred_element_type=jnp.float32)
    h2 = _bn_relu(z2 + vec_ref[3:4, :], vec_ref[4:5, :], vec_ref[5:6, :])
    h_ref[...] = h2
    t_ref[...] = jnp.dot(h2, wl_ref[...], preferred_element_type=jnp.float32) + vec_ref[6:7, :]


def _stage_conv_body(hin_ref, a_ref, w1_ref, w2_ref, wl_ref, vec_ref, h_ref, t_ref):
    x = hin_ref[...] + a_ref[0, 0:N, :] + a_ref[1, 0:N, :]
    z = jnp.dot(x, w1_ref[...], preferred_element_type=jnp.float32)
    h = _bn_relu(z + vec_ref[0:1, :], vec_ref[1:2, :], vec_ref[2:3, :])
    z2 = jnp.dot(h, w2_ref[...], preferred_element_type=jnp.float32)
    h2 = _bn_relu(z2 + vec_ref[3:4, :], vec_ref[4:5, :], vec_ref[5:6, :])
    h_ref[...] = h2
    t_ref[...] = jnp.dot(h2, wl_ref[...], preferred_element_type=jnp.float32) + vec_ref[6:7, :]


_STAGE_OUT = (
    jax.ShapeDtypeStruct((N, H), jnp.float32),
    jax.ShapeDtypeStruct((N, T), jnp.float32),
)


def _pack_vecs(mlp, lin):
    return jnp.stack([
        mlp["lin1"]["b"], mlp["bn1"]["g"], mlp["bn1"]["b"],
        mlp["lin2"]["b"], mlp["bn2"]["g"], mlp["bn2"]["b"],
        lin["b"], jnp.zeros((H,), jnp.float32),
    ])


def _stage_first(x, mlp, lin):
    return pl.pallas_call(_stage_first_body, out_shape=_STAGE_OUT)(
        x, mlp["lin1"]["W"].T, mlp["lin2"]["W"].T, lin["W"].T,
        _pack_vecs(mlp, lin))


def _stage_conv(hin, agg, mlp, lin):
    return pl.pallas_call(_stage_conv_body, out_shape=_STAGE_OUT)(
        hin, agg, mlp["lin1"]["W"].T, mlp["lin2"]["W"].T, lin["W"].T,
        _pack_vecs(mlp, lin))


def _pool_body(t_ref, b_ref, o_ref):
    gblk = pl.program_id(0)
    t = t_ref[...]
    bids = b_ref[...]
    for k in range(8):
        mask = bids == gblk * 8 + k
        o_ref[k:k + 1, :] = jnp.max(jnp.where(mask, t, -jnp.inf), axis=0,
                                    keepdims=True)


def _pool(t, bids):
    return pl.pallas_call(
        _pool_body,
        grid=(G // 8,),
        in_specs=[
            pl.BlockSpec((N, T), lambda g: (0, 0)),
            pl.BlockSpec((N, T), lambda g: (0, 0)),
        ],
        out_specs=pl.BlockSpec((8, T), lambda g: (g, 0)),
        out_shape=jax.ShapeDtypeStruct((G, T), jnp.float32),
    )(t, bids)


def kernel(x, edge_index, batch, params):
    ei = edge_index.astype(jnp.int32)
    pad = EP - E
    parange = jnp.arange(pad, dtype=jnp.int32)
    src = jnp.concatenate([ei[0], parange % N])
    dst = jnp.concatenate([ei[1], N + parange % (NP - N)])
    srcs = src.reshape(NW, CH, CHUNK)
    dsts = dst.reshape(NW, CH, CHUNK)
    zeros = jnp.zeros((NP, H), jnp.float32)
    bids = jnp.broadcast_to(batch.astype(jnp.int32)[:, None], (N, T))

    p = params
    h, t = _stage_first(x, p["first_h"], p["linears"][0])
    out = _pool(t, bids)
    for i in range(2):
        agg = _sc_segsum(h, srcs, dsts, zeros)
        h, t = _stage_conv(h, agg, p["convs"][i], p["linears"][i + 1])
        out = out + _pool(t, bids)
    return (out, h)


# speedup vs baseline: 7.6854x; 1.0668x over previous
import functools

import jax
import jax.numpy as jnp
from jax import lax
from jax.experimental import pallas as pl
from jax.experimental.pallas import tpu as pltpu
from jax.experimental.pallas import tpu_sc as plsc

N = 10000
E = 320000
D = 128
H = 64
T = 64
G = 64

NC = 2
NS = 16
NW = NC * NS

CHUNK = 128
CH = 80
EPT = CH * CHUNK
EP = NW * EPT
NP = 10112
RPT = NP // NS


NB = 4
LA = 2


def _sc_segsum_body(h_hbm, srcs_hbm, dsts_hbm, zeros_hbm, out_hbm,
                    src_v, dst_v, bufs, gsems, ssems, agg_sh):
    c = lax.axis_index("c")
    s = lax.axis_index("s")
    wid = s * NC + c

    pltpu.sync_copy(srcs_hbm.at[wid], src_v)
    pltpu.sync_copy(dsts_hbm.at[wid], dst_v)

    r0 = s * RPT
    pltpu.sync_copy(zeros_hbm.at[pl.ds(r0, RPT)], agg_sh.at[pl.ds(r0, RPT)])
    plsc.subcore_barrier()

    def start_gather(j, b):
        pltpu.async_copy(h_hbm.at[src_v.at[j]], bufs[b], gsems[b])

    def wait_gather(j, b):
        pltpu.make_async_copy(h_hbm.at[src_v.at[j]], bufs[b], gsems[b]).wait()

    def start_scatter(j, b):
        pltpu.async_copy(bufs[b], agg_sh.at[dst_v.at[j]], ssems[b], add=True)

    def wait_scatter(j, b):
        pltpu.make_async_copy(bufs[b], agg_sh.at[dst_v.at[j]], ssems[b]).wait()

    for j in range(LA):
        start_gather(j, j % NB)

    @pl.loop(0, CH, step=NB)
    def _(jj):
        for b in range(NB):
            j = jj + b
            bg = (b + LA) % NB

            @pl.when(j + LA < CH)
            def _():
                @pl.when(j + LA - NB >= 0)
                def _():
                    wait_scatter(j + LA - NB, bg)
                start_gather(j + LA, bg)

            wait_gather(j, b)
            start_scatter(j, b)

    for k in range(CH - NB, CH):
        wait_scatter(k, k % NB)

    plsc.subcore_barrier()
    pltpu.sync_copy(agg_sh.at[pl.ds(r0, RPT)], out_hbm.at[c, pl.ds(r0, RPT)])


def _sc_segsum(h, srcs, dsts, zeros):
    mesh = plsc.VectorSubcoreMesh(core_axis_name="c", subcore_axis_name="s")
    return pl.kernel(
        _sc_segsum_body,
        out_type=jax.ShapeDtypeStruct((NC, NP, H), jnp.float32),
        mesh=mesh,
        scratch_types=[
            pltpu.VMEM((CH, CHUNK), jnp.int32),
            pltpu.VMEM((CH, CHUNK), jnp.int32),
            [pltpu.VMEM((CHUNK, H), jnp.float32) for _ in range(NB)],
            [pltpu.SemaphoreType.DMA for _ in range(NB)],
            [pltpu.SemaphoreType.DMA for _ in range(NB)],
            pltpu.VMEM_SHARED((NP, H), jnp.float32),
        ],
        compiler_params=pltpu.CompilerParams(use_tc_tiling_on_sc=False),
        name="sc_segsum",
    )(h, srcs, dsts, zeros)


def _bn_relu(z, g, b):
    m = jnp.mean(z, axis=0, keepdims=True)
    d = z - m
    v = jnp.mean(d * d, axis=0, keepdims=True)
    return jnp.maximum(d * lax.rsqrt(v + 1e-5) * g + b, 0.0)


def _stage_first_body(x_ref, w1_ref, w2_ref, wl_ref, vec_ref, h_ref, t_ref):
    x = x_ref[...]
    z = jnp.dot(x, w1_ref[...], preferred_element_type=jnp.float32)
    h = _bn_relu(z + vec_ref[0:1, :], vec_ref[1:2, :], vec_ref[2:3, :])
    z2 = jnp.dot(h, w2_ref[...], preferred_element_type=jnp.float32)
    h2 = _bn_relu(z2 + vec_ref[3:4, :], vec_ref[4:5, :], vec_ref[5:6, :])
    h_ref[...] = h2
    t_ref[...] = jnp.dot(h2, wl_ref[...], preferred_element_type=jnp.float32) + vec_ref[6:7, :]


def _stage_conv_body(hin_ref, a_ref, w1_ref, w2_ref, wl_ref, vec_ref, h_ref, t_ref):
    x = hin_ref[...] + a_ref[0, 0:N, :] + a_ref[1, 0:N, :]
    z = jnp.dot(x, w1_ref[...], preferred_element_type=jnp.float32)
    h = _bn_relu(z + vec_ref[0:1, :], vec_ref[1:2, :], vec_ref[2:3, :])
    z2 = jnp.dot(h, w2_ref[...], preferred_element_type=jnp.float32)
    h2 = _bn_relu(z2 + vec_ref[3:4, :], vec_ref[4:5, :], vec_ref[5:6, :])
    h_ref[...] = h2
    t_ref[...] = jnp.dot(h2, wl_ref[...], preferred_element_type=jnp.float32) + vec_ref[6:7, :]


_STAGE_OUT = (
    jax.ShapeDtypeStruct((N, H), jnp.float32),
    jax.ShapeDtypeStruct((N, T), jnp.float32),
)


def _pack_vecs(mlp, lin):
    return jnp.stack([
        mlp["lin1"]["b"], mlp["bn1"]["g"], mlp["bn1"]["b"],
        mlp["lin2"]["b"], mlp["bn2"]["g"], mlp["bn2"]["b"],
        lin["b"], jnp.zeros((H,), jnp.float32),
    ])


def _stage_first(x, mlp, lin):
    return pl.pallas_call(_stage_first_body, out_shape=_STAGE_OUT)(
        x, mlp["lin1"]["W"].T, mlp["lin2"]["W"].T, lin["W"].T,
        _pack_vecs(mlp, lin))


def _stage_conv(hin, agg, mlp, lin):
    return pl.pallas_call(_stage_conv_body, out_shape=_STAGE_OUT)(
        hin, agg, mlp["lin1"]["W"].T, mlp["lin2"]["W"].T, lin["W"].T,
        _pack_vecs(mlp, lin))


def _pool_body(t_ref, b_ref, o_ref):
    gblk = pl.program_id(0)
    t = t_ref[...].astype(jnp.bfloat16)
    bids = b_ref[...]
    neg = jnp.float32(-jnp.inf).astype(jnp.bfloat16)
    for k in range(8):
        mask = bids == (gblk * 8 + k).astype(jnp.bfloat16)
        m = jnp.max(jnp.where(mask, t, neg), axis=0, keepdims=True)
        o_ref[k:k + 1, :] = m.astype(jnp.float32)


def _pool(t, bids):
    return pl.pallas_call(
        _pool_body,
        grid=(G // 8,),
        in_specs=[
            pl.BlockSpec((N, T), lambda g: (0, 0)),
            pl.BlockSpec((N, T), lambda g: (0, 0)),
        ],
        out_specs=pl.BlockSpec((8, T), lambda g: (g, 0)),
        out_shape=jax.ShapeDtypeStruct((G, T), jnp.float32),
    )(t, bids)


def kernel(x, edge_index, batch, params):
    ei = edge_index.astype(jnp.int32)
    pad = EP - E
    parange = jnp.arange(pad, dtype=jnp.int32)
    src = jnp.concatenate([ei[0], parange % N])
    dst = jnp.concatenate([ei[1], N + parange % (NP - N)])
    srcs = src.reshape(NW, CH, CHUNK)
    dsts = dst.reshape(NW, CH, CHUNK)
    zeros = jnp.zeros((NP, H), jnp.float32)
    bids = jnp.broadcast_to(batch.astype(jnp.bfloat16)[:, None], (N, T))

    p = params
    h, t = _stage_first(x, p["first_h"], p["linears"][0])
    out = _pool(t, bids)
    for i in range(2):
        agg = _sc_segsum(h, srcs, dsts, zeros)
        h, t = _stage_conv(h, agg, p["convs"][i], p["linears"][i + 1])
        out = out + _pool(t, bids)
    return (out, h)
